# baseline (speedup 1.0000x reference)
import functools

import jax
import jax.numpy as jnp
from jax import lax
from jax.experimental import pallas as pl
from jax.experimental.pallas import tpu as pltpu
from jax.experimental.pallas import tpu_sc as plsc

NC, NS, L = 2, 16, 16
NW = NC * NS


def _make_sc_add(B, S, D, rows_per_tile):
    rows_per_worker = S // NW
    n_tiles = rows_per_worker // rows_per_tile
    elems = rows_per_tile * D

    mesh = plsc.VectorSubcoreMesh(
        core_axis_name="c", subcore_axis_name="s",
        num_cores=NC, num_subcores=NS)

    @functools.partial(
        pl.kernel,
        out_type=jax.ShapeDtypeStruct((B, S * D), jnp.float32),
        mesh=mesh,
        scratch_types=[
            pltpu.VMEM((elems,), jnp.float32),
            pltpu.VMEM((elems,), jnp.float32),
        ],
    )
    def sc_add(in_hbm, tbl_hbm, out_hbm, tbl_v, io_v):
        wid = lax.axis_index("s") * NC + lax.axis_index("c")
        base = wid * (rows_per_worker * D)
        for t in range(n_tiles):
            off = base + t * elems
            pltpu.sync_copy(tbl_hbm.at[pl.ds(off, elems)], tbl_v)
            for b in range(B):
                pltpu.sync_copy(in_hbm.at[b, pl.ds(off, elems)], io_v)

                def body(i, carry):
                    sl = pl.ds(i * L, L)
                    io_v[sl] = io_v[sl] + tbl_v[sl]
                    return carry

                lax.fori_loop(0, elems // L, body, 0)
                pltpu.sync_copy(io_v, out_hbm.at[b, pl.ds(off, elems)])

    return sc_add


def kernel(inputs, pos_table):
    B, S, D = inputs.shape
    sc_add = _make_sc_add(B, S, D, rows_per_tile=32)
    out = sc_add(inputs.reshape(B, S * D), pos_table.reshape(S * D))
    return out.reshape(B, S, D)


# speedup vs baseline: 1.8121x; 1.8121x over previous
import functools

import jax
import jax.numpy as jnp
from jax import lax
from jax.experimental import pallas as pl
from jax.experimental.pallas import tpu as pltpu
from jax.experimental.pallas import tpu_sc as plsc

NC, NS, L = 2, 16, 16
NW = NC * NS


def _make_sc_add(B, S, D, rows_per_tile):
    rows_per_worker = S // NW
    n_tiles = rows_per_worker // rows_per_tile
    elems = rows_per_tile * D
    n_jobs = n_tiles * B

    mesh = plsc.VectorSubcoreMesh(
        core_axis_name="c", subcore_axis_name="s",
        num_cores=NC, num_subcores=NS)

    @functools.partial(
        pl.kernel,
        out_type=jax.ShapeDtypeStruct((B, S * D), jnp.float32),
        mesh=mesh,
        scratch_types=[
            [pltpu.VMEM((elems,), jnp.float32) for _ in range(2)],
            [pltpu.VMEM((elems,), jnp.float32) for _ in range(2)],
            [pltpu.SemaphoreType.DMA for _ in range(2)],
            [pltpu.SemaphoreType.DMA for _ in range(2)],
            [pltpu.SemaphoreType.DMA for _ in range(2)],
        ],
    )
    def sc_add(in_hbm, tbl_hbm, out_hbm, io_v, tbl_v, in_sem, tbl_sem, out_sem):
        wid = lax.axis_index("s") * NC + lax.axis_index("c")
        base = wid * (rows_per_worker * D)

        def start_in(j):
            t, b = divmod(j, B)
            off = base + t * elems
            return pltpu.async_copy(
                in_hbm.at[b, pl.ds(off, elems)], io_v[j % 2], in_sem[j % 2])

        def start_tbl(t):
            off = base + t * elems
            return pltpu.async_copy(
                tbl_hbm.at[pl.ds(off, elems)], tbl_v[t % 2], tbl_sem[t % 2])

        h_tbl = {0: start_tbl(0)}
        h_in = {0: start_in(0)}
        h_out = {}
        for j in range(n_jobs):
            t, b = divmod(j, B)
            cur = j % 2
            if j + 1 < n_jobs:
                t1, b1 = divmod(j + 1, B)
                if b1 == 0:
                    h_tbl[t1] = start_tbl(t1)
                if j - 1 in h_out:
                    h_out.pop(j - 1).wait()
                h_in[j + 1] = start_in(j + 1)
            if b == 0:
                h_tbl.pop(t).wait()
            h_in.pop(j).wait()

            tbl_buf = tbl_v[t % 2]
            io_buf = io_v[cur]

            @plsc.parallel_loop(0, elems, step=L, unroll=8)
            def _(i):
                plsc.addupdate(io_buf.at[pl.ds(i, L)], tbl_buf[pl.ds(i, L)])

            off = base + t * elems
            h_out[j] = pltpu.async_copy(
                io_buf, out_hbm.at[b, pl.ds(off, elems)], out_sem[cur])
        for j in sorted(h_out):
            h_out.pop(j).wait()

    return sc_add


def kernel(inputs, pos_table):
    B, S, D = inputs.shape
    sc_add = _make_sc_add(B, S, D, rows_per_tile=16)
    out = sc_add(inputs.reshape(B, S * D), pos_table.reshape(S * D))
    return out.reshape(B, S, D)


# speedup vs baseline: 4.4814x; 2.4730x over previous
import functools

import jax
import jax.numpy as jnp
from jax import lax
from jax.experimental import pallas as pl
from jax.experimental.pallas import tpu as pltpu
from jax.experimental.pallas import tpu_sc as plsc

NC, NS, L = 2, 16, 16
NW = NC * NS


def _make_sc_add(B, S, D, rows_per_tile):
    rows_per_worker = S // NW
    n_tiles = rows_per_worker // rows_per_tile
    elems = rows_per_tile * D
    n_jobs = n_tiles * B
    shift = D.bit_length() - 1

    mesh = plsc.VectorSubcoreMesh(
        core_axis_name="c", subcore_axis_name="s",
        num_cores=NC, num_subcores=NS)

    @functools.partial(
        pl.kernel,
        out_type=jax.ShapeDtypeStruct((B, S, D), jnp.float32),
        mesh=mesh,
        scratch_types=[
            [pltpu.VMEM((rows_per_tile, D), jnp.float32) for _ in range(2)],
            [pltpu.VMEM((rows_per_tile, D), jnp.float32) for _ in range(2)],
            [pltpu.SemaphoreType.DMA for _ in range(2)],
            [pltpu.SemaphoreType.DMA for _ in range(2)],
            [pltpu.SemaphoreType.DMA for _ in range(2)],
        ],
    )
    def sc_add(in_hbm, tbl_hbm, out_hbm, io_v, tbl_v, in_sem, tbl_sem, out_sem):
        wid = lax.axis_index("s") * NC + lax.axis_index("c")
        base = wid * rows_per_worker

        def start_in(j):
            t, b = divmod(j, B)
            r0 = base + t * rows_per_tile
            return pltpu.async_copy(
                in_hbm.at[b, pl.ds(r0, rows_per_tile), :],
                io_v[j % 2], in_sem[j % 2])

        def start_tbl(t):
            r0 = base + t * rows_per_tile
            return pltpu.async_copy(
                tbl_hbm.at[pl.ds(r0, rows_per_tile), :],
                tbl_v[t % 2], tbl_sem[t % 2])

        h_tbl = {0: start_tbl(0)}
        h_in = {0: start_in(0)}
        h_out = {}
        for j in range(n_jobs):
            t, b = divmod(j, B)
            cur = j % 2
            if j + 1 < n_jobs:
                t1, b1 = divmod(j + 1, B)
                if b1 == 0:
                    h_tbl[t1] = start_tbl(t1)
                if j - 1 in h_out:
                    h_out.pop(j - 1).wait()
                h_in[j + 1] = start_in(j + 1)
            if b == 0:
                h_tbl.pop(t).wait()
            h_in.pop(j).wait()

            tbl_buf = tbl_v[t % 2]
            io_buf = io_v[cur]

            @plsc.parallel_loop(0, elems, step=L, unroll=8)
            def _(i):
                r = lax.shift_right_logical(i, shift)
                c = pl.multiple_of(lax.bitwise_and(i, D - 1), L)
                plsc.addupdate(io_buf.at[r, pl.ds(c, L)],
                               tbl_buf[r, pl.ds(c, L)])

            r0 = base + t * rows_per_tile
            h_out[j] = pltpu.async_copy(
                io_buf, out_hbm.at[b, pl.ds(r0, rows_per_tile), :],
                out_sem[cur])
        for j in sorted(h_out):
            h_out.pop(j).wait()

    return sc_add


def kernel(inputs, pos_table):
    B, S, D = inputs.shape
    sc_add = _make_sc_add(B, S, D, rows_per_tile=16)
    return sc_add(inputs, pos_table)
